# Initial kernel scaffold; baseline (speedup 1.0000x reference)
import jax
import jax.numpy as jnp
from jax.experimental import pallas as pl


def kernel(x, pitch_bins, energy_bins, pitch_embedding, energy_embedding):
    raise NotImplementedError("write your pallas kernel here")


# speedup vs baseline: 96.7529x; 96.7529x over previous
import functools

import jax
import jax.numpy as jnp
from jax.experimental import pallas as pl


def _body(x_ref, plo_ref, phi_ref, elo_ref, ehi_ref, ptab_ref, etab_ref,
          out_ref):
    vp = x_ref[0, 0:1, :]
    ve = x_ref[0, 1:2, :]
    oh_p = ((plo_ref[:, :] < vp) & (phi_ref[:, :] >= vp)).astype(jnp.float32)
    oh_e = ((elo_ref[:, :] < ve) & (ehi_ref[:, :] >= ve)).astype(jnp.float32)
    out_ref[0] = (
        jnp.dot(ptab_ref[:, :], oh_p, preferred_element_type=jnp.float32)
        + jnp.dot(etab_ref[:, :], oh_e, preferred_element_type=jnp.float32)
    )


@functools.partial(jax.jit, static_argnames=("interpret",))
def kernel(x, pitch_bins, energy_bins, pitch_embedding, energy_embedding,
           interpret=False):
    B, _, T = x.shape
    N, H = pitch_embedding.shape
    Tt = 512

    inf = jnp.array([jnp.inf], dtype=jnp.float32)
    p_lo = jnp.concatenate([-inf, pitch_bins])[:, None]
    p_hi = jnp.concatenate([pitch_bins, inf])[:, None]
    e_lo = jnp.concatenate([-inf, energy_bins])[:, None]
    e_hi = jnp.concatenate([energy_bins, inf])[:, None]
    ptab = pitch_embedding.T
    etab = energy_embedding.T

    grid = (B, T // Tt)
    return pl.pallas_call(
        _body,
        grid=grid,
        in_specs=[
            pl.BlockSpec((1, 2, Tt), lambda b, j: (b, 0, j)),
            pl.BlockSpec((N, 1), lambda b, j: (0, 0)),
            pl.BlockSpec((N, 1), lambda b, j: (0, 0)),
            pl.BlockSpec((N, 1), lambda b, j: (0, 0)),
            pl.BlockSpec((N, 1), lambda b, j: (0, 0)),
            pl.BlockSpec((H, N), lambda b, j: (0, 0)),
            pl.BlockSpec((H, N), lambda b, j: (0, 0)),
        ],
        out_specs=pl.BlockSpec((1, H, Tt), lambda b, j: (b, 0, j)),
        out_shape=jax.ShapeDtypeStruct((B, H, T), jnp.float32),
        interpret=interpret,
    )(x, p_lo, p_hi, e_lo, e_hi, ptab, etab)


# speedup vs baseline: 97.1931x; 1.0045x over previous
import functools

import jax
import jax.numpy as jnp
from jax.experimental import pallas as pl


def _body(x_ref, plo_ref, phi_ref, elo_ref, ehi_ref, ptab_ref, etab_ref,
          out_ref):
    vp = x_ref[0, 0:1, :]
    ve = x_ref[0, 1:2, :]
    oh_p = ((plo_ref[:, :] < vp) & (phi_ref[:, :] >= vp)).astype(jnp.bfloat16)
    oh_e = ((elo_ref[:, :] < ve) & (ehi_ref[:, :] >= ve)).astype(jnp.bfloat16)
    out_ref[0] = (
        jnp.dot(ptab_ref[:, :], oh_p, preferred_element_type=jnp.float32)
        + jnp.dot(etab_ref[:, :], oh_e, preferred_element_type=jnp.float32)
    )


@functools.partial(jax.jit, static_argnames=("interpret",))
def kernel(x, pitch_bins, energy_bins, pitch_embedding, energy_embedding,
           interpret=False):
    B, _, T = x.shape
    N, H = pitch_embedding.shape
    Tt = 512

    inf = jnp.array([jnp.inf], dtype=jnp.float32)
    p_lo = jnp.concatenate([-inf, pitch_bins])[:, None]
    p_hi = jnp.concatenate([pitch_bins, inf])[:, None]
    e_lo = jnp.concatenate([-inf, energy_bins])[:, None]
    e_hi = jnp.concatenate([energy_bins, inf])[:, None]
    ptab = pitch_embedding.T.astype(jnp.bfloat16)
    etab = energy_embedding.T.astype(jnp.bfloat16)

    grid = (B, T // Tt)
    return pl.pallas_call(
        _body,
        grid=grid,
        in_specs=[
            pl.BlockSpec((1, 2, Tt), lambda b, j: (b, 0, j)),
            pl.BlockSpec((N, 1), lambda b, j: (0, 0)),
            pl.BlockSpec((N, 1), lambda b, j: (0, 0)),
            pl.BlockSpec((N, 1), lambda b, j: (0, 0)),
            pl.BlockSpec((N, 1), lambda b, j: (0, 0)),
            pl.BlockSpec((H, N), lambda b, j: (0, 0)),
            pl.BlockSpec((H, N), lambda b, j: (0, 0)),
        ],
        out_specs=pl.BlockSpec((1, H, Tt), lambda b, j: (b, 0, j)),
        out_shape=jax.ShapeDtypeStruct((B, H, T), jnp.float32),
        interpret=interpret,
    )(x, p_lo, p_hi, e_lo, e_hi, ptab, etab)
